# Initial kernel scaffold; baseline (speedup 1.0000x reference)
import jax
import jax.numpy as jnp
from jax.experimental import pallas as pl


def kernel(user_emb, item_emb, adj_vals, u_mul_s, v_mul_s, ut, vt, adj_rows, adj_cols):
    raise NotImplementedError("write your pallas kernel here")


# speedup vs baseline: 3.2552x; 3.2552x over previous
import functools

import jax
import jax.numpy as jnp
from jax import lax
from jax.experimental import pallas as pl
from jax.experimental.pallas import tpu as pltpu
from jax.experimental.pallas import tpu_sc as plsc

NU = 25000
NI = 25000
D = 64
RK = 32
NNZ = 800000

NTILE = 16
NCORE = 2
NPAD = 25088
ROWS_PER_TILE = NPAD // NTILE
ZCHUNK = 224
CHUNK = 128
EPT = 50048
NNZ_PAD = EPT * NTILE
NCHUNK = EPT // CHUNK


def _spmm_body(ego_u, ego_i, vals, rows, cols, zu_out, zi_out,
               gidx, sidx, valb, gbuf, zbuf, accum, sem):
    cid = lax.axis_index("c")
    sid = lax.axis_index("s")

    zero16 = jnp.zeros((16,), jnp.float32)

    def zrow(e, carry):
        for j in range(4):
            zbuf[e, pl.ds(j * 16, 16)] = zero16
        return carry

    lax.fori_loop(0, ZCHUNK, zrow, 0)
    for k in range(ROWS_PER_TILE // ZCHUNK):
        pltpu.sync_copy(zbuf, accum.at[pl.ds(sid * ROWS_PER_TILE + k * ZCHUNK, ZCHUNK)])
    plsc.subcore_barrier()

    def run_side(src_tbl, g_hbm, s_hbm, out_hbm):
        def chunk_body(c, carry):
            base = sid * EPT + c * CHUNK
            pltpu.sync_copy(g_hbm.at[pl.ds(base, CHUNK)], gidx)
            pltpu.sync_copy(s_hbm.at[pl.ds(base, CHUNK)], sidx)
            pltpu.sync_copy(vals.at[pl.ds(base, CHUNK)], valb)
            pltpu.async_copy(src_tbl.at[gidx], gbuf, sem).wait()

            def group_body(g, c2):
                vv = valb[pl.ds(g * 16, 16)]
                for i in range(16):
                    v = vv[i]
                    e = g * 16 + i
                    for j in range(4):
                        sl = pl.ds(j * 16, 16)
                        gbuf[e, sl] = gbuf[e, sl] * v
                return c2

            lax.fori_loop(0, CHUNK // 16, group_body, 0)
            pltpu.sync_copy(gbuf, accum.at[sidx], add=True)
            return carry

        lax.fori_loop(0, NCHUNK, chunk_body, 0)
        plsc.subcore_barrier()
        for k in range(ROWS_PER_TILE // ZCHUNK):
            off = sid * ROWS_PER_TILE + k * ZCHUNK
            pltpu.sync_copy(accum.at[pl.ds(off, ZCHUNK)],
                            out_hbm.at[pl.ds(off, ZCHUNK)])

    @pl.when(cid == 0)
    def _():
        run_side(ego_i, cols, rows, zu_out)

    @pl.when(cid == 1)
    def _():
        run_side(ego_u, rows, cols, zi_out)


@jax.jit
def _spmm(ego_u_pad, ego_i_pad, vals_pad, rows_pad, cols_pad):
    mesh = plsc.VectorSubcoreMesh(core_axis_name="c", subcore_axis_name="s",
                                  num_cores=NCORE, num_subcores=NTILE)
    f = pl.kernel(
        _spmm_body,
        out_type=[jax.ShapeDtypeStruct((NPAD, D), jnp.float32),
                  jax.ShapeDtypeStruct((NPAD, D), jnp.float32)],
        mesh=mesh,
        scratch_types=[
            pltpu.VMEM((CHUNK,), jnp.int32),
            pltpu.VMEM((CHUNK,), jnp.int32),
            pltpu.VMEM((CHUNK,), jnp.float32),
            pltpu.VMEM((CHUNK, D), jnp.float32),
            pltpu.VMEM((ZCHUNK, D), jnp.float32),
            pltpu.VMEM_SHARED((NPAD, D), jnp.float32),
            pltpu.SemaphoreType.DMA,
        ],
        compiler_params=pltpu.CompilerParams(use_tc_tiling_on_sc=False),
        name="lightgcl_spmm",
    )
    return f(ego_u_pad, ego_i_pad, vals_pad, rows_pad, cols_pad)


KBLK = 1792


def _reduce_body(vt_ref, ut_ref, ie_ref, zi1_ref, ue_ref, zu1_ref, su_ref, si_ref):
    @pl.when(pl.program_id(0) == 0)
    def _():
        su_ref[...] = jnp.zeros_like(su_ref)
        si_ref[...] = jnp.zeros_like(si_ref)

    su_ref[...] += jnp.dot(vt_ref[...], (ie_ref[...] + zi1_ref[...]) * 0.5,
                           preferred_element_type=jnp.float32)
    si_ref[...] += jnp.dot(ut_ref[...], (ue_ref[...] + zu1_ref[...]) * 0.5,
                           preferred_element_type=jnp.float32)


@jax.jit
def _tc_reduce(vt_pad, ut_pad, ie_pad, zi1, ue_pad, zu1):
    grid = NPAD // KBLK
    return pl.pallas_call(
        _reduce_body,
        grid=(grid,),
        in_specs=[
            pl.BlockSpec((RK, KBLK), lambda k: (0, k)),
            pl.BlockSpec((RK, KBLK), lambda k: (0, k)),
            pl.BlockSpec((KBLK, D), lambda k: (k, 0)),
            pl.BlockSpec((KBLK, D), lambda k: (k, 0)),
            pl.BlockSpec((KBLK, D), lambda k: (k, 0)),
            pl.BlockSpec((KBLK, D), lambda k: (k, 0)),
        ],
        out_specs=[
            pl.BlockSpec((RK, D), lambda k: (0, 0)),
            pl.BlockSpec((RK, D), lambda k: (0, 0)),
        ],
        out_shape=[jax.ShapeDtypeStruct((RK, D), jnp.float32),
                   jax.ShapeDtypeStruct((RK, D), jnp.float32)],
    )(vt_pad, ut_pad, ie_pad, zi1, ue_pad, zu1)


def _bcast_body(um_ref, vm_ref, su_ref, si_ref, zu1_ref, zu2_ref, zi1_ref, zi2_ref,
                ua_ref, ia_ref, ug_ref, ig_ref):
    ua_ref[...] = (zu1_ref[...] + zu2_ref[...]) * 0.5
    ia_ref[...] = (zi1_ref[...] + zi2_ref[...]) * 0.5
    ug_ref[...] = jnp.dot(um_ref[...], su_ref[...],
                          preferred_element_type=jnp.float32)
    ig_ref[...] = jnp.dot(vm_ref[...], si_ref[...],
                          preferred_element_type=jnp.float32)


@jax.jit
def _tc_bcast(um_pad, vm_pad, su, si, zu1, zu2, zi1, zi2):
    grid = NPAD // KBLK
    node_spec = pl.BlockSpec((KBLK, D), lambda k: (k, 0))
    full_spec = pl.BlockSpec((RK, D), lambda k: (0, 0))
    return pl.pallas_call(
        _bcast_body,
        grid=(grid,),
        in_specs=[
            pl.BlockSpec((KBLK, RK), lambda k: (k, 0)),
            pl.BlockSpec((KBLK, RK), lambda k: (k, 0)),
            full_spec, full_spec,
            node_spec, node_spec, node_spec, node_spec,
        ],
        out_specs=[node_spec, node_spec, node_spec, node_spec],
        out_shape=[jax.ShapeDtypeStruct((NPAD, D), jnp.float32)] * 4,
    )(um_pad, vm_pad, su, si, zu1, zu2, zi1, zi2)


def kernel(user_emb, item_emb, adj_vals, u_mul_s, v_mul_s, ut, vt, adj_rows, adj_cols):
    ue_pad = jnp.pad(user_emb, ((0, NPAD - NU), (0, 0)))
    ie_pad = jnp.pad(item_emb, ((0, NPAD - NI), (0, 0)))
    vals_pad = jnp.pad(adj_vals, (0, NNZ_PAD - NNZ))
    rows_pad = jnp.pad(adj_rows, (0, NNZ_PAD - NNZ))
    cols_pad = jnp.pad(adj_cols, (0, NNZ_PAD - NNZ))
    vt_pad = jnp.pad(vt, ((0, 0), (0, NPAD - NI)))
    ut_pad = jnp.pad(ut, ((0, 0), (0, NPAD - NU)))
    um_pad = jnp.pad(u_mul_s, ((0, NPAD - NU), (0, 0)))
    vm_pad = jnp.pad(v_mul_s, ((0, NPAD - NI), (0, 0)))

    zu1, zi1 = _spmm(ue_pad, ie_pad, vals_pad, rows_pad, cols_pad)
    zu2, zi2 = _spmm(zu1, zi1, vals_pad, rows_pad, cols_pad)
    su, si = _tc_reduce(vt_pad, ut_pad, ie_pad, zi1, ue_pad, zu1)
    ua, ia, ug, ig = _tc_bcast(um_pad, vm_pad, su, si, zu1, zu2, zi1, zi2)
    return (ua[:NU], ia[:NI], ug[:NU], ig[:NI])


# speedup vs baseline: 5.7278x; 1.7596x over previous
import functools

import jax
import jax.numpy as jnp
from jax import lax
from jax.experimental import pallas as pl
from jax.experimental.pallas import tpu as pltpu
from jax.experimental.pallas import tpu_sc as plsc

NU = 25000
NI = 25000
D = 64
RK = 32
NNZ = 800000

NTILE = 16
NCORE = 2
NPAD = 25088
ROWS_PER_TILE = NPAD // NTILE
ZCHUNK = 56
CHUNK = 128
EPT = 50048
NNZ_PAD = EPT * NTILE
NCHUNK = EPT // CHUNK
BLKC = 17
NBLK = NCHUNK // BLKC
NBUF = 3
NIDX = 4


def _spmm_body(ego_u, ego_i, vals, rows, cols, zu_out, zi_out,
               gidxc, sidxc, valbc, gbufs, zbuf, accum,
               gsems, ssems, xsems, isems, vsems, zsem):
    cid = lax.axis_index("c")
    sid = lax.axis_index("s")

    zero16 = jnp.zeros((16,), jnp.float32)

    def zrow(e, carry):
        for j in range(4):
            zbuf[e, pl.ds(j * 16, 16)] = zero16
        return carry

    lax.fori_loop(0, ZCHUNK, zrow, 0)
    zds = []
    for k in range(ROWS_PER_TILE // ZCHUNK):
        zds.append(pltpu.async_copy(
            zbuf, accum.at[pl.ds(sid * ROWS_PER_TILE + k * ZCHUNK, ZCHUNK)],
            zsem))
    for d in zds:
        d.wait()
    plsc.subcore_barrier()

    def run_side(src_tbl, g_hbm, s_hbm, out_hbm):
        def block_body(b, carry):
            ebase = (sid * NCHUNK + b * BLKC) * CHUNK

            def load_idx(j):
                gx = pltpu.async_copy(
                    g_hbm.at[pl.ds(ebase + j * CHUNK, CHUNK)],
                    gidxc[j % NBUF], xsems[j % NBUF])
                si = pltpu.async_copy(
                    s_hbm.at[pl.ds(ebase + j * CHUNK, CHUNK)],
                    sidxc[j % NIDX], isems[j % NIDX])
                vl = pltpu.async_copy(
                    vals.at[pl.ds(ebase + j * CHUNK, CHUNK)],
                    valbc[j % NIDX], vsems[j % NIDX])
                return gx, si, vl

            gxds = [None] * BLKC
            ids = [None] * BLKC
            vds = [None] * BLKC
            gds = [None] * BLKC
            sds = [None] * BLKC
            gxds[0], ids[0], vds[0] = load_idx(0)
            for j in range(BLKC + 2):
                if j >= 3 and j - 3 < BLKC:
                    sds[j - 3].wait()
                if j >= 2:
                    i = j - 2
                    buf = gbufs[i % NBUF]
                    vds[i].wait()
                    ids[i].wait()
                    gds[i].wait()
                    vb = valbc[i % NIDX]

                    def group_body(g, c2):
                        vv = vb[pl.ds(g * 16, 16)]
                        for t in range(16):
                            v = vv[t]
                            e = g * 16 + t
                            for q in range(4):
                                sl = pl.ds(q * 16, 16)
                                buf[e, sl] = buf[e, sl] * v
                        return c2

                    lax.fori_loop(0, CHUNK // 16, group_body, 0)
                    sds[i] = pltpu.async_copy(buf, accum.at[sidxc[i % NIDX]],
                                              ssems[i % NBUF], add=True)
                if j < BLKC:
                    gxds[j].wait()
                    gds[j] = pltpu.async_copy(
                        src_tbl.at[gidxc[j % NBUF]], gbufs[j % NBUF],
                        gsems[j % NBUF])
                if j + 1 < BLKC:
                    gxds[j + 1], ids[j + 1], vds[j + 1] = load_idx(j + 1)
            sds[BLKC - 1].wait()
            return carry

        lax.fori_loop(0, NBLK, block_body, 0)
        plsc.subcore_barrier()
        for k in range(ROWS_PER_TILE // ZCHUNK):
            off = sid * ROWS_PER_TILE + k * ZCHUNK
            pltpu.sync_copy(accum.at[pl.ds(off, ZCHUNK)],
                            out_hbm.at[pl.ds(off, ZCHUNK)])

    @pl.when(cid == 0)
    def _():
        run_side(ego_i, cols, rows, zu_out)

    @pl.when(cid == 1)
    def _():
        run_side(ego_u, rows, cols, zi_out)


@jax.jit
def _spmm(ego_u_pad, ego_i_pad, vals_pad, rows_pad, cols_pad):
    mesh = plsc.VectorSubcoreMesh(core_axis_name="c", subcore_axis_name="s",
                                  num_cores=NCORE, num_subcores=NTILE)
    f = pl.kernel(
        _spmm_body,
        out_type=[jax.ShapeDtypeStruct((NPAD, D), jnp.float32),
                  jax.ShapeDtypeStruct((NPAD, D), jnp.float32)],
        mesh=mesh,
        scratch_types=[
            [pltpu.VMEM((CHUNK,), jnp.int32)] * NBUF,
            [pltpu.VMEM((CHUNK,), jnp.int32)] * NIDX,
            [pltpu.VMEM((CHUNK,), jnp.float32)] * NIDX,
            [pltpu.VMEM((CHUNK, D), jnp.float32)] * NBUF,
            pltpu.VMEM((ZCHUNK, D), jnp.float32),
            pltpu.VMEM_SHARED((NPAD, D), jnp.float32),
            [pltpu.SemaphoreType.DMA] * NBUF,
            [pltpu.SemaphoreType.DMA] * NBUF,
            [pltpu.SemaphoreType.DMA] * NBUF,
            [pltpu.SemaphoreType.DMA] * NIDX,
            [pltpu.SemaphoreType.DMA] * NIDX,
            pltpu.SemaphoreType.DMA,
        ],
        compiler_params=pltpu.CompilerParams(use_tc_tiling_on_sc=False),
        name="lightgcl_spmm",
    )
    return f(ego_u_pad, ego_i_pad, vals_pad, rows_pad, cols_pad)


KBLK = 1792


def _reduce_body(vt_ref, ut_ref, ie_ref, zi1_ref, ue_ref, zu1_ref, su_ref, si_ref):
    @pl.when(pl.program_id(0) == 0)
    def _():
        su_ref[...] = jnp.zeros_like(su_ref)
        si_ref[...] = jnp.zeros_like(si_ref)

    su_ref[...] += jnp.dot(vt_ref[...], (ie_ref[...] + zi1_ref[...]) * 0.5,
                           preferred_element_type=jnp.float32)
    si_ref[...] += jnp.dot(ut_ref[...], (ue_ref[...] + zu1_ref[...]) * 0.5,
                           preferred_element_type=jnp.float32)


@jax.jit
def _tc_reduce(vt_pad, ut_pad, ie_pad, zi1, ue_pad, zu1):
    grid = NPAD // KBLK
    return pl.pallas_call(
        _reduce_body,
        grid=(grid,),
        in_specs=[
            pl.BlockSpec((RK, KBLK), lambda k: (0, k)),
            pl.BlockSpec((RK, KBLK), lambda k: (0, k)),
            pl.BlockSpec((KBLK, D), lambda k: (k, 0)),
            pl.BlockSpec((KBLK, D), lambda k: (k, 0)),
            pl.BlockSpec((KBLK, D), lambda k: (k, 0)),
            pl.BlockSpec((KBLK, D), lambda k: (k, 0)),
        ],
        out_specs=[
            pl.BlockSpec((RK, D), lambda k: (0, 0)),
            pl.BlockSpec((RK, D), lambda k: (0, 0)),
        ],
        out_shape=[jax.ShapeDtypeStruct((RK, D), jnp.float32),
                   jax.ShapeDtypeStruct((RK, D), jnp.float32)],
    )(vt_pad, ut_pad, ie_pad, zi1, ue_pad, zu1)


def _bcast_body(um_ref, vm_ref, su_ref, si_ref, zu1_ref, zu2_ref, zi1_ref, zi2_ref,
                ua_ref, ia_ref, ug_ref, ig_ref):
    ua_ref[...] = (zu1_ref[...] + zu2_ref[...]) * 0.5
    ia_ref[...] = (zi1_ref[...] + zi2_ref[...]) * 0.5
    ug_ref[...] = jnp.dot(um_ref[...], su_ref[...],
                          preferred_element_type=jnp.float32)
    ig_ref[...] = jnp.dot(vm_ref[...], si_ref[...],
                          preferred_element_type=jnp.float32)


@jax.jit
def _tc_bcast(um_pad, vm_pad, su, si, zu1, zu2, zi1, zi2):
    grid = NPAD // KBLK
    node_spec = pl.BlockSpec((KBLK, D), lambda k: (k, 0))
    full_spec = pl.BlockSpec((RK, D), lambda k: (0, 0))
    return pl.pallas_call(
        _bcast_body,
        grid=(grid,),
        in_specs=[
            pl.BlockSpec((KBLK, RK), lambda k: (k, 0)),
            pl.BlockSpec((KBLK, RK), lambda k: (k, 0)),
            full_spec, full_spec,
            node_spec, node_spec, node_spec, node_spec,
        ],
        out_specs=[node_spec, node_spec, node_spec, node_spec],
        out_shape=[jax.ShapeDtypeStruct((NPAD, D), jnp.float32)] * 4,
    )(um_pad, vm_pad, su, si, zu1, zu2, zi1, zi2)


def kernel(user_emb, item_emb, adj_vals, u_mul_s, v_mul_s, ut, vt, adj_rows, adj_cols):
    ue_pad = jnp.pad(user_emb, ((0, NPAD - NU), (0, 0)))
    ie_pad = jnp.pad(item_emb, ((0, NPAD - NI), (0, 0)))
    vals_pad = jnp.pad(adj_vals, (0, NNZ_PAD - NNZ))
    rows_pad = jnp.pad(adj_rows, (0, NNZ_PAD - NNZ))
    cols_pad = jnp.pad(adj_cols, (0, NNZ_PAD - NNZ))
    vt_pad = jnp.pad(vt, ((0, 0), (0, NPAD - NI)))
    ut_pad = jnp.pad(ut, ((0, 0), (0, NPAD - NU)))
    um_pad = jnp.pad(u_mul_s, ((0, NPAD - NU), (0, 0)))
    vm_pad = jnp.pad(v_mul_s, ((0, NPAD - NI), (0, 0)))

    zu1, zi1 = _spmm(ue_pad, ie_pad, vals_pad, rows_pad, cols_pad)
    zu2, zi2 = _spmm(zu1, zi1, vals_pad, rows_pad, cols_pad)
    su, si = _tc_reduce(vt_pad, ut_pad, ie_pad, zi1, ue_pad, zu1)
    ua, ia, ug, ig = _tc_bcast(um_pad, vm_pad, su, si, zu1, zu2, zi1, zi2)
    return (ua[:NU], ia[:NI], ug[:NU], ig[:NI])


# speedup vs baseline: 7.0422x; 1.2295x over previous
import functools

import jax
import jax.numpy as jnp
from jax import lax
from jax.experimental import pallas as pl
from jax.experimental.pallas import tpu as pltpu
from jax.experimental.pallas import tpu_sc as plsc

NU = 25000
NI = 25000
D = 64
RK = 32
NNZ = 800000

NTILE = 16
NCORE = 2
NPAD = 25088
ROWS_PER_TILE = NPAD // NTILE
ZCHUNK = 16
CHUNK = 128
EPT = 50176
NNZ_PAD = EPT * NTILE
NCHUNK = EPT // CHUNK
BLKC = 8
NBLK = NCHUNK // BLKC
NBUF = 3
NIDX = 4
NVAL = 2


def _spmm_body(ego_u, ego_i, vals, rows, cols, zu_out, zi_out,
               gidxc, sidxc, valbc, gbufs, zbuf, accum,
               gsems, ssems, xsems, isems, vsems, zsem):
    cid = lax.axis_index("c")
    sid = lax.axis_index("s")

    zero16 = jnp.zeros((16,), jnp.float32)

    def zrow(e, carry):
        for j in range(4):
            zbuf[e, pl.ds(j * 16, 16)] = zero16
        return carry

    lax.fori_loop(0, ZCHUNK, zrow, 0)
    zds = []
    for k in range(ROWS_PER_TILE // ZCHUNK):
        zds.append(pltpu.async_copy(
            zbuf, accum.at[pl.ds(sid * ROWS_PER_TILE + k * ZCHUNK, ZCHUNK)],
            zsem))
    for d in zds:
        d.wait()
    plsc.subcore_barrier()

    def run_side(src_tbl, g_hbm, s_hbm, out_hbm):
        def block_body(b, carry):
            ebase = (sid * NCHUNK + b * BLKC) * CHUNK

            def load_idx(j):
                gx = pltpu.async_copy(
                    g_hbm.at[pl.ds(ebase + j * CHUNK, CHUNK)],
                    gidxc[j % NBUF], xsems[j % NBUF])
                si = pltpu.async_copy(
                    s_hbm.at[pl.ds(ebase + j * CHUNK, CHUNK)],
                    sidxc[j % NIDX], isems[j % NIDX])
                return gx, si

            gxds = [None] * BLKC
            ids = [None] * BLKC
            vds = [None] * BLKC
            gds = [None] * BLKC
            sds = [None] * BLKC
            gxds[0], ids[0] = load_idx(0)
            for j in range(BLKC + 2):
                if j >= 3 and j - 3 < BLKC:
                    sds[j - 3].wait()
                if j >= 2:
                    i = j - 2
                    buf = gbufs[i % NBUF]
                    vds[i].wait()
                    ids[i].wait()
                    gds[i].wait()
                    vb = valbc[i % NVAL]

                    def group_body(g, c2):
                        for t in range(16):
                            e = g * 16 + t
                            v16 = vb[e]
                            for q in range(4):
                                sl = pl.ds(q * 16, 16)
                                buf[e, sl] = buf[e, sl] * v16
                        return c2

                    lax.fori_loop(0, CHUNK // 16, group_body, 0)
                    sds[i] = pltpu.async_copy(buf, accum.at[sidxc[i % NIDX]],
                                              ssems[i % NBUF], add=True)
                if j < BLKC:
                    gxds[j].wait()
                    gds[j] = pltpu.async_copy(
                        src_tbl.at[gidxc[j % NBUF]], gbufs[j % NBUF],
                        gsems[j % NBUF])
                    vds[j] = pltpu.async_copy(
                        vals.at[pl.ds(ebase + j * CHUNK, CHUNK)],
                        valbc[j % NVAL], vsems[j % NVAL])
                if j + 1 < BLKC:
                    gxds[j + 1], ids[j + 1] = load_idx(j + 1)
            sds[BLKC - 1].wait()
            return carry

        lax.fori_loop(0, NBLK, block_body, 0)
        plsc.subcore_barrier()
        cpd = pltpu.async_copy(
            accum.at[pl.ds(sid * ROWS_PER_TILE, ROWS_PER_TILE)],
            out_hbm.at[pl.ds(sid * ROWS_PER_TILE, ROWS_PER_TILE)], zsem)
        cpd.wait()

    @pl.when(cid == 0)
    def _():
        run_side(ego_i, cols, rows, zu_out)

    @pl.when(cid == 1)
    def _():
        run_side(ego_u, rows, cols, zi_out)


@jax.jit
def _spmm(ego_u_pad, ego_i_pad, vals_pad, rows_pad, cols_pad):
    mesh = plsc.VectorSubcoreMesh(core_axis_name="c", subcore_axis_name="s",
                                  num_cores=NCORE, num_subcores=NTILE)
    f = pl.kernel(
        _spmm_body,
        out_type=[jax.ShapeDtypeStruct((NPAD, D), jnp.float32),
                  jax.ShapeDtypeStruct((NPAD, D), jnp.float32)],
        mesh=mesh,
        scratch_types=[
            [pltpu.VMEM((CHUNK,), jnp.int32)] * NBUF,
            [pltpu.VMEM((CHUNK,), jnp.int32)] * NIDX,
            [pltpu.VMEM((CHUNK, 16), jnp.float32)] * NVAL,
            [pltpu.VMEM((CHUNK, D), jnp.float32)] * NBUF,
            pltpu.VMEM((ZCHUNK, D), jnp.float32),
            pltpu.VMEM_SHARED((NPAD, D), jnp.float32),
            [pltpu.SemaphoreType.DMA] * NBUF,
            [pltpu.SemaphoreType.DMA] * NBUF,
            [pltpu.SemaphoreType.DMA] * NBUF,
            [pltpu.SemaphoreType.DMA] * NIDX,
            [pltpu.SemaphoreType.DMA] * NVAL,
            pltpu.SemaphoreType.DMA,
        ],
        compiler_params=pltpu.CompilerParams(use_tc_tiling_on_sc=False),
        name="lightgcl_spmm",
    )
    return f(ego_u_pad, ego_i_pad, vals_pad, rows_pad, cols_pad)


KBLK = 1792


def _reduce_body(vt_ref, ut_ref, ie_ref, zi1_ref, ue_ref, zu1_ref, su_ref, si_ref):
    @pl.when(pl.program_id(0) == 0)
    def _():
        su_ref[...] = jnp.zeros_like(su_ref)
        si_ref[...] = jnp.zeros_like(si_ref)

    su_ref[...] += jnp.dot(vt_ref[...], (ie_ref[...] + zi1_ref[...]) * 0.5,
                           preferred_element_type=jnp.float32)
    si_ref[...] += jnp.dot(ut_ref[...], (ue_ref[...] + zu1_ref[...]) * 0.5,
                           preferred_element_type=jnp.float32)


@jax.jit
def _tc_reduce(vt_pad, ut_pad, ie_pad, zi1, ue_pad, zu1):
    grid = NPAD // KBLK
    return pl.pallas_call(
        _reduce_body,
        grid=(grid,),
        in_specs=[
            pl.BlockSpec((RK, KBLK), lambda k: (0, k)),
            pl.BlockSpec((RK, KBLK), lambda k: (0, k)),
            pl.BlockSpec((KBLK, D), lambda k: (k, 0)),
            pl.BlockSpec((KBLK, D), lambda k: (k, 0)),
            pl.BlockSpec((KBLK, D), lambda k: (k, 0)),
            pl.BlockSpec((KBLK, D), lambda k: (k, 0)),
        ],
        out_specs=[
            pl.BlockSpec((RK, D), lambda k: (0, 0)),
            pl.BlockSpec((RK, D), lambda k: (0, 0)),
        ],
        out_shape=[jax.ShapeDtypeStruct((RK, D), jnp.float32),
                   jax.ShapeDtypeStruct((RK, D), jnp.float32)],
    )(vt_pad, ut_pad, ie_pad, zi1, ue_pad, zu1)


def _bcast_body(um_ref, vm_ref, su_ref, si_ref, zu1_ref, zu2_ref, zi1_ref, zi2_ref,
                ua_ref, ia_ref, ug_ref, ig_ref):
    ua_ref[...] = (zu1_ref[...] + zu2_ref[...]) * 0.5
    ia_ref[...] = (zi1_ref[...] + zi2_ref[...]) * 0.5
    ug_ref[...] = jnp.dot(um_ref[...], su_ref[...],
                          preferred_element_type=jnp.float32)
    ig_ref[...] = jnp.dot(vm_ref[...], si_ref[...],
                          preferred_element_type=jnp.float32)


@jax.jit
def _tc_bcast(um_pad, vm_pad, su, si, zu1, zu2, zi1, zi2):
    grid = NPAD // KBLK
    node_spec = pl.BlockSpec((KBLK, D), lambda k: (k, 0))
    full_spec = pl.BlockSpec((RK, D), lambda k: (0, 0))
    return pl.pallas_call(
        _bcast_body,
        grid=(grid,),
        in_specs=[
            pl.BlockSpec((KBLK, RK), lambda k: (k, 0)),
            pl.BlockSpec((KBLK, RK), lambda k: (k, 0)),
            full_spec, full_spec,
            node_spec, node_spec, node_spec, node_spec,
        ],
        out_specs=[node_spec, node_spec, node_spec, node_spec],
        out_shape=[jax.ShapeDtypeStruct((NPAD, D), jnp.float32)] * 4,
    )(um_pad, vm_pad, su, si, zu1, zu2, zi1, zi2)


def kernel(user_emb, item_emb, adj_vals, u_mul_s, v_mul_s, ut, vt, adj_rows, adj_cols):
    ue_pad = jnp.pad(user_emb, ((0, NPAD - NU), (0, 0)))
    ie_pad = jnp.pad(item_emb, ((0, NPAD - NI), (0, 0)))
    vals_pad = jnp.pad(adj_vals, (0, NNZ_PAD - NNZ))
    rows_pad = jnp.pad(adj_rows, (0, NNZ_PAD - NNZ))
    cols_pad = jnp.pad(adj_cols, (0, NNZ_PAD - NNZ))
    vt_pad = jnp.pad(vt, ((0, 0), (0, NPAD - NI)))
    ut_pad = jnp.pad(ut, ((0, 0), (0, NPAD - NU)))
    um_pad = jnp.pad(u_mul_s, ((0, NPAD - NU), (0, 0)))
    vm_pad = jnp.pad(v_mul_s, ((0, NPAD - NI), (0, 0)))

    vals_rep = jnp.broadcast_to(vals_pad[:, None], (NNZ_PAD, 16))

    zu1, zi1 = _spmm(ue_pad, ie_pad, vals_rep, rows_pad, cols_pad)
    zu2, zi2 = _spmm(zu1, zi1, vals_rep, rows_pad, cols_pad)
    su, si = _tc_reduce(vt_pad, ut_pad, ie_pad, zi1, ue_pad, zu1)
    ua, ia, ug, ig = _tc_bcast(um_pad, vm_pad, su, si, zu1, zu2, zi1, zi2)
    return (ua[:NU], ia[:NI], ug[:NU], ig[:NI])


# speedup vs baseline: 7.8548x; 1.1154x over previous
import functools

import jax
import jax.numpy as jnp
from jax import lax
from jax.experimental import pallas as pl
from jax.experimental.pallas import tpu as pltpu
from jax.experimental.pallas import tpu_sc as plsc

NU = 25000
NI = 25000
D = 64
RK = 32
NNZ = 800000

NTILE = 16
NCORE = 2
NPAD = 25088
ROWS_PER_TILE = NPAD // NTILE
ZCHUNK = 16
CHUNK = 128
EPT = 50176
NNZ_PAD = EPT * NTILE
NCHUNK = EPT // CHUNK
BLKC = 8
NBLK = NCHUNK // BLKC
NBUF = 3
NIDX = 4
NVAL = 2


def _spmm_body(ego_u, ego_i, vals, rows, cols, zu_out, zi_out,
               gidxc, sidxc, valbc, gbufs, zbuf, accum,
               gsems, ssems, xsems, isems, vsems, zsem):
    cid = lax.axis_index("c")
    sid = lax.axis_index("s")

    zero16 = jnp.zeros((16,), jnp.float32)

    def zrow(e, carry):
        for j in range(4):
            zbuf[e, pl.ds(j * 16, 16)] = zero16
        return carry

    lax.fori_loop(0, ZCHUNK, zrow, 0)
    zds = []
    for k in range(ROWS_PER_TILE // ZCHUNK):
        zds.append(pltpu.async_copy(
            zbuf, accum.at[pl.ds(sid * ROWS_PER_TILE + k * ZCHUNK, ZCHUNK)],
            zsem))
    for d in zds:
        d.wait()
    plsc.subcore_barrier()

    def run_side(src_tbl, g_hbm, s_hbm, out_hbm):
        def block_body(b, carry):
            ebase = (sid * NCHUNK + b * BLKC) * CHUNK

            def load_idx(j):
                gx = pltpu.async_copy(
                    g_hbm.at[pl.ds(ebase + j * CHUNK, CHUNK)],
                    gidxc[j % NBUF], xsems[j % NBUF])
                si = pltpu.async_copy(
                    s_hbm.at[pl.ds(ebase + j * CHUNK, CHUNK)],
                    sidxc[j % NIDX], isems[j % NIDX])
                return gx, si

            gxds = [None] * BLKC
            ids = [None] * BLKC
            vds = [None] * BLKC
            gds = [None] * BLKC
            sds = [None] * BLKC
            gxds[0], ids[0] = load_idx(0)
            for j in range(BLKC + 2):
                if j >= 3 and j - 3 < BLKC:
                    sds[j - 3].wait()
                if j >= 2:
                    i = j - 2
                    buf = gbufs[i % NBUF]
                    vds[i].wait()
                    ids[i].wait()
                    gds[i].wait()
                    vb = valbc[i % NVAL]

                    def group_body(g, c2):
                        for t in range(0, 16, 2):
                            e0 = g * 16 + t
                            e1 = e0 + 1
                            va = vb[e0]
                            vc = vb[e1]
                            for q in range(4):
                                sl = pl.ds(q * 16, 16)
                                buf[e0, sl] = buf[e0, sl] * va
                                buf[e1, sl] = buf[e1, sl] * vc
                        return c2

                    lax.fori_loop(0, CHUNK // 16, group_body, 0)
                    sds[i] = pltpu.async_copy(buf, accum.at[sidxc[i % NIDX]],
                                              ssems[i % NBUF], add=True)
                if j < BLKC:
                    gxds[j].wait()
                    gds[j] = pltpu.async_copy(
                        src_tbl.at[gidxc[j % NBUF]], gbufs[j % NBUF],
                        gsems[j % NBUF])
                    vds[j] = pltpu.async_copy(
                        vals.at[pl.ds(ebase + j * CHUNK, CHUNK)],
                        valbc[j % NVAL], vsems[j % NVAL])
                if j + 1 < BLKC:
                    gxds[j + 1], ids[j + 1] = load_idx(j + 1)
            sds[BLKC - 1].wait()
            return carry

        lax.fori_loop(0, NBLK, block_body, 0)
        plsc.subcore_barrier()
        cpd = pltpu.async_copy(
            accum.at[pl.ds(sid * ROWS_PER_TILE, ROWS_PER_TILE)],
            out_hbm.at[pl.ds(sid * ROWS_PER_TILE, ROWS_PER_TILE)], zsem)
        cpd.wait()

    @pl.when(cid == 0)
    def _():
        run_side(ego_i, cols, rows, zu_out)

    @pl.when(cid == 1)
    def _():
        run_side(ego_u, rows, cols, zi_out)


@jax.jit
def _spmm(ego_u_pad, ego_i_pad, vals_pad, rows_pad, cols_pad):
    mesh = plsc.VectorSubcoreMesh(core_axis_name="c", subcore_axis_name="s",
                                  num_cores=NCORE, num_subcores=NTILE)
    f = pl.kernel(
        _spmm_body,
        out_type=[jax.ShapeDtypeStruct((NPAD, D), jnp.float32),
                  jax.ShapeDtypeStruct((NPAD, D), jnp.float32)],
        mesh=mesh,
        scratch_types=[
            [pltpu.VMEM((CHUNK,), jnp.int32)] * NBUF,
            [pltpu.VMEM((CHUNK,), jnp.int32)] * NIDX,
            [pltpu.VMEM((CHUNK, 16), jnp.float32)] * NVAL,
            [pltpu.VMEM((CHUNK, D), jnp.float32)] * NBUF,
            pltpu.VMEM((ZCHUNK, D), jnp.float32),
            pltpu.VMEM_SHARED((NPAD, D), jnp.float32),
            [pltpu.SemaphoreType.DMA] * NBUF,
            [pltpu.SemaphoreType.DMA] * NBUF,
            [pltpu.SemaphoreType.DMA] * NBUF,
            [pltpu.SemaphoreType.DMA] * NIDX,
            [pltpu.SemaphoreType.DMA] * NVAL,
            pltpu.SemaphoreType.DMA,
        ],
        compiler_params=pltpu.CompilerParams(use_tc_tiling_on_sc=False),
        name="lightgcl_spmm",
    )
    return f(ego_u_pad, ego_i_pad, vals_pad, rows_pad, cols_pad)


KBLK = 1792


def _reduce_body(vt_ref, ut_ref, ie_ref, zi1_ref, ue_ref, zu1_ref, su_ref, si_ref):
    @pl.when(pl.program_id(0) == 0)
    def _():
        su_ref[...] = jnp.zeros_like(su_ref)
        si_ref[...] = jnp.zeros_like(si_ref)

    su_ref[...] += jnp.dot(vt_ref[...], (ie_ref[...] + zi1_ref[...]) * 0.5,
                           preferred_element_type=jnp.float32)
    si_ref[...] += jnp.dot(ut_ref[...], (ue_ref[...] + zu1_ref[...]) * 0.5,
                           preferred_element_type=jnp.float32)


@jax.jit
def _tc_reduce(vt_pad, ut_pad, ie_pad, zi1, ue_pad, zu1):
    grid = NPAD // KBLK
    return pl.pallas_call(
        _reduce_body,
        grid=(grid,),
        in_specs=[
            pl.BlockSpec((RK, KBLK), lambda k: (0, k)),
            pl.BlockSpec((RK, KBLK), lambda k: (0, k)),
            pl.BlockSpec((KBLK, D), lambda k: (k, 0)),
            pl.BlockSpec((KBLK, D), lambda k: (k, 0)),
            pl.BlockSpec((KBLK, D), lambda k: (k, 0)),
            pl.BlockSpec((KBLK, D), lambda k: (k, 0)),
        ],
        out_specs=[
            pl.BlockSpec((RK, D), lambda k: (0, 0)),
            pl.BlockSpec((RK, D), lambda k: (0, 0)),
        ],
        out_shape=[jax.ShapeDtypeStruct((RK, D), jnp.float32),
                   jax.ShapeDtypeStruct((RK, D), jnp.float32)],
    )(vt_pad, ut_pad, ie_pad, zi1, ue_pad, zu1)


def _bcast_body(um_ref, vm_ref, su_ref, si_ref, zu1_ref, zu2_ref, zi1_ref, zi2_ref,
                ua_ref, ia_ref, ug_ref, ig_ref):
    ua_ref[...] = (zu1_ref[...] + zu2_ref[...]) * 0.5
    ia_ref[...] = (zi1_ref[...] + zi2_ref[...]) * 0.5
    ug_ref[...] = jnp.dot(um_ref[...], su_ref[...],
                          preferred_element_type=jnp.float32)
    ig_ref[...] = jnp.dot(vm_ref[...], si_ref[...],
                          preferred_element_type=jnp.float32)


@jax.jit
def _tc_bcast(um_pad, vm_pad, su, si, zu1, zu2, zi1, zi2):
    grid = NPAD // KBLK
    node_spec = pl.BlockSpec((KBLK, D), lambda k: (k, 0))
    full_spec = pl.BlockSpec((RK, D), lambda k: (0, 0))
    return pl.pallas_call(
        _bcast_body,
        grid=(grid,),
        in_specs=[
            pl.BlockSpec((KBLK, RK), lambda k: (k, 0)),
            pl.BlockSpec((KBLK, RK), lambda k: (k, 0)),
            full_spec, full_spec,
            node_spec, node_spec, node_spec, node_spec,
        ],
        out_specs=[node_spec, node_spec, node_spec, node_spec],
        out_shape=[jax.ShapeDtypeStruct((NPAD, D), jnp.float32)] * 4,
    )(um_pad, vm_pad, su, si, zu1, zu2, zi1, zi2)


def kernel(user_emb, item_emb, adj_vals, u_mul_s, v_mul_s, ut, vt, adj_rows, adj_cols):
    ue_pad = jnp.pad(user_emb, ((0, NPAD - NU), (0, 0)))
    ie_pad = jnp.pad(item_emb, ((0, NPAD - NI), (0, 0)))
    vals_pad = jnp.pad(adj_vals, (0, NNZ_PAD - NNZ))
    rows_pad = jnp.pad(adj_rows, (0, NNZ_PAD - NNZ))
    cols_pad = jnp.pad(adj_cols, (0, NNZ_PAD - NNZ))
    vt_pad = jnp.pad(vt, ((0, 0), (0, NPAD - NI)))
    ut_pad = jnp.pad(ut, ((0, 0), (0, NPAD - NU)))
    um_pad = jnp.pad(u_mul_s, ((0, NPAD - NU), (0, 0)))
    vm_pad = jnp.pad(v_mul_s, ((0, NPAD - NI), (0, 0)))

    vals_rep = jnp.broadcast_to(vals_pad[:, None], (NNZ_PAD, 16))

    zu1, zi1 = _spmm(ue_pad, ie_pad, vals_rep, rows_pad, cols_pad)
    zu2, zi2 = _spmm(zu1, zi1, vals_rep, rows_pad, cols_pad)
    su, si = _tc_reduce(vt_pad, ut_pad, ie_pad, zi1, ue_pad, zu1)
    ua, ia, ug, ig = _tc_bcast(um_pad, vm_pad, su, si, zu1, zu2, zi1, zi2)
    return (ua[:NU], ia[:NI], ug[:NU], ig[:NI])


# speedup vs baseline: 8.2758x; 1.0536x over previous
import functools

import jax
import jax.numpy as jnp
from jax import lax
from jax.experimental import pallas as pl
from jax.experimental.pallas import tpu as pltpu
from jax.experimental.pallas import tpu_sc as plsc

NU = 25000
NI = 25000
D = 64
RK = 32
NNZ = 800000

NTILE = 16
NCORE = 2
NPAD = 25088
ROWS_PER_TILE = NPAD // NTILE
ZCHUNK = 16
CHUNK = 128
EPT = 50176
NNZ_PAD = EPT * NTILE
NCHUNK = EPT // CHUNK
BLKC = 8
NBLK = NCHUNK // BLKC
NBUF = 3
NIDX = 4
NVAL = 2


def _spmm_body(ego_u, ego_i, vals, rows, cols, zu_out, zi_out,
               gidxc, sidxc, valbc, gbufs, zbuf, accum,
               gsems, ssems, xsems, isems, vsems, zsem):
    cid = lax.axis_index("c")
    sid = lax.axis_index("s")

    zero16 = jnp.zeros((16,), jnp.float32)

    def zrow(e, carry):
        for j in range(4):
            zbuf[e, pl.ds(j * 16, 16)] = zero16
        return carry

    lax.fori_loop(0, ZCHUNK, zrow, 0)
    zds = []
    for k in range(ROWS_PER_TILE // ZCHUNK):
        zds.append(pltpu.async_copy(
            zbuf, accum.at[pl.ds(sid * ROWS_PER_TILE + k * ZCHUNK, ZCHUNK)],
            zsem))
    for d in zds:
        d.wait()
    plsc.subcore_barrier()

    def run_side(src_tbl, g_hbm, s_hbm, out_hbm):
        def block_body(b, carry):
            ebase = (sid * NCHUNK + b * BLKC) * CHUNK

            def load_idx(j):
                gx = pltpu.async_copy(
                    g_hbm.at[pl.ds(ebase + j * CHUNK, CHUNK)],
                    gidxc[j % NBUF], xsems[j % NBUF])
                si = pltpu.async_copy(
                    s_hbm.at[pl.ds(ebase + j * CHUNK, CHUNK)],
                    sidxc[j % NIDX], isems[j % NIDX])
                return gx, si

            gxds = [None] * BLKC
            ids = [None] * BLKC
            vds = [None] * BLKC
            gds = [None] * BLKC
            sds = [None] * BLKC
            gxds[0], ids[0] = load_idx(0)
            for j in range(BLKC + 2):
                if j >= 3 and j - 3 < BLKC:
                    sds[j - 3].wait()
                if j >= 2:
                    i = j - 2
                    buf = gbufs[i % NBUF]
                    vds[i].wait()
                    ids[i].wait()
                    gds[i].wait()
                    vb = valbc[i % NVAL]

                    def group_body(g, c2):
                        for t in range(0, 16, 4):
                            e0 = g * 16 + t
                            vv = [vb[e0 + k] for k in range(4)]
                            for q in range(4):
                                sl = pl.ds(q * 16, 16)
                                for k in range(4):
                                    buf[e0 + k, sl] = buf[e0 + k, sl] * vv[k]
                        return c2

                    lax.fori_loop(0, CHUNK // 16, group_body, 0)
                    sds[i] = pltpu.async_copy(buf, accum.at[sidxc[i % NIDX]],
                                              ssems[i % NBUF], add=True)
                if j < BLKC:
                    gxds[j].wait()
                    gds[j] = pltpu.async_copy(
                        src_tbl.at[gidxc[j % NBUF]], gbufs[j % NBUF],
                        gsems[j % NBUF])
                    vds[j] = pltpu.async_copy(
                        vals.at[pl.ds(ebase + j * CHUNK, CHUNK)],
                        valbc[j % NVAL], vsems[j % NVAL])
                if j + 1 < BLKC:
                    gxds[j + 1], ids[j + 1] = load_idx(j + 1)
            sds[BLKC - 1].wait()
            return carry

        lax.fori_loop(0, NBLK, block_body, 0)
        plsc.subcore_barrier()
        cpd = pltpu.async_copy(
            accum.at[pl.ds(sid * ROWS_PER_TILE, ROWS_PER_TILE)],
            out_hbm.at[pl.ds(sid * ROWS_PER_TILE, ROWS_PER_TILE)], zsem)
        cpd.wait()

    @pl.when(cid == 0)
    def _():
        run_side(ego_i, cols, rows, zu_out)

    @pl.when(cid == 1)
    def _():
        run_side(ego_u, rows, cols, zi_out)


@jax.jit
def _spmm(ego_u_pad, ego_i_pad, vals_pad, rows_pad, cols_pad):
    mesh = plsc.VectorSubcoreMesh(core_axis_name="c", subcore_axis_name="s",
                                  num_cores=NCORE, num_subcores=NTILE)
    f = pl.kernel(
        _spmm_body,
        out_type=[jax.ShapeDtypeStruct((NPAD, D), jnp.float32),
                  jax.ShapeDtypeStruct((NPAD, D), jnp.float32)],
        mesh=mesh,
        scratch_types=[
            [pltpu.VMEM((CHUNK,), jnp.int32)] * NBUF,
            [pltpu.VMEM((CHUNK,), jnp.int32)] * NIDX,
            [pltpu.VMEM((CHUNK, 16), jnp.float32)] * NVAL,
            [pltpu.VMEM((CHUNK, D), jnp.float32)] * NBUF,
            pltpu.VMEM((ZCHUNK, D), jnp.float32),
            pltpu.VMEM_SHARED((NPAD, D), jnp.float32),
            [pltpu.SemaphoreType.DMA] * NBUF,
            [pltpu.SemaphoreType.DMA] * NBUF,
            [pltpu.SemaphoreType.DMA] * NBUF,
            [pltpu.SemaphoreType.DMA] * NIDX,
            [pltpu.SemaphoreType.DMA] * NVAL,
            pltpu.SemaphoreType.DMA,
        ],
        compiler_params=pltpu.CompilerParams(use_tc_tiling_on_sc=False),
        name="lightgcl_spmm",
    )
    return f(ego_u_pad, ego_i_pad, vals_pad, rows_pad, cols_pad)


KBLK = 1792


def _reduce_body(vt_ref, ut_ref, ie_ref, zi1_ref, ue_ref, zu1_ref, su_ref, si_ref):
    @pl.when(pl.program_id(0) == 0)
    def _():
        su_ref[...] = jnp.zeros_like(su_ref)
        si_ref[...] = jnp.zeros_like(si_ref)

    su_ref[...] += jnp.dot(vt_ref[...], (ie_ref[...] + zi1_ref[...]) * 0.5,
                           preferred_element_type=jnp.float32)
    si_ref[...] += jnp.dot(ut_ref[...], (ue_ref[...] + zu1_ref[...]) * 0.5,
                           preferred_element_type=jnp.float32)


@jax.jit
def _tc_reduce(vt_pad, ut_pad, ie_pad, zi1, ue_pad, zu1):
    grid = NPAD // KBLK
    return pl.pallas_call(
        _reduce_body,
        grid=(grid,),
        in_specs=[
            pl.BlockSpec((RK, KBLK), lambda k: (0, k)),
            pl.BlockSpec((RK, KBLK), lambda k: (0, k)),
            pl.BlockSpec((KBLK, D), lambda k: (k, 0)),
            pl.BlockSpec((KBLK, D), lambda k: (k, 0)),
            pl.BlockSpec((KBLK, D), lambda k: (k, 0)),
            pl.BlockSpec((KBLK, D), lambda k: (k, 0)),
        ],
        out_specs=[
            pl.BlockSpec((RK, D), lambda k: (0, 0)),
            pl.BlockSpec((RK, D), lambda k: (0, 0)),
        ],
        out_shape=[jax.ShapeDtypeStruct((RK, D), jnp.float32),
                   jax.ShapeDtypeStruct((RK, D), jnp.float32)],
    )(vt_pad, ut_pad, ie_pad, zi1, ue_pad, zu1)


def _bcast_body(um_ref, vm_ref, su_ref, si_ref, zu1_ref, zu2_ref, zi1_ref, zi2_ref,
                ua_ref, ia_ref, ug_ref, ig_ref):
    ua_ref[...] = (zu1_ref[...] + zu2_ref[...]) * 0.5
    ia_ref[...] = (zi1_ref[...] + zi2_ref[...]) * 0.5
    ug_ref[...] = jnp.dot(um_ref[...], su_ref[...],
                          preferred_element_type=jnp.float32)
    ig_ref[...] = jnp.dot(vm_ref[...], si_ref[...],
                          preferred_element_type=jnp.float32)


@jax.jit
def _tc_bcast(um_pad, vm_pad, su, si, zu1, zu2, zi1, zi2):
    grid = NPAD // KBLK
    node_spec = pl.BlockSpec((KBLK, D), lambda k: (k, 0))
    full_spec = pl.BlockSpec((RK, D), lambda k: (0, 0))
    return pl.pallas_call(
        _bcast_body,
        grid=(grid,),
        in_specs=[
            pl.BlockSpec((KBLK, RK), lambda k: (k, 0)),
            pl.BlockSpec((KBLK, RK), lambda k: (k, 0)),
            full_spec, full_spec,
            node_spec, node_spec, node_spec, node_spec,
        ],
        out_specs=[node_spec, node_spec, node_spec, node_spec],
        out_shape=[jax.ShapeDtypeStruct((NPAD, D), jnp.float32)] * 4,
    )(um_pad, vm_pad, su, si, zu1, zu2, zi1, zi2)


def kernel(user_emb, item_emb, adj_vals, u_mul_s, v_mul_s, ut, vt, adj_rows, adj_cols):
    ue_pad = jnp.pad(user_emb, ((0, NPAD - NU), (0, 0)))
    ie_pad = jnp.pad(item_emb, ((0, NPAD - NI), (0, 0)))
    vals_pad = jnp.pad(adj_vals, (0, NNZ_PAD - NNZ))
    rows_pad = jnp.pad(adj_rows, (0, NNZ_PAD - NNZ))
    cols_pad = jnp.pad(adj_cols, (0, NNZ_PAD - NNZ))
    vt_pad = jnp.pad(vt, ((0, 0), (0, NPAD - NI)))
    ut_pad = jnp.pad(ut, ((0, 0), (0, NPAD - NU)))
    um_pad = jnp.pad(u_mul_s, ((0, NPAD - NU), (0, 0)))
    vm_pad = jnp.pad(v_mul_s, ((0, NPAD - NI), (0, 0)))

    vals_rep = jnp.broadcast_to(vals_pad[:, None], (NNZ_PAD, 16))

    zu1, zi1 = _spmm(ue_pad, ie_pad, vals_rep, rows_pad, cols_pad)
    zu2, zi2 = _spmm(zu1, zi1, vals_rep, rows_pad, cols_pad)
    su, si = _tc_reduce(vt_pad, ut_pad, ie_pad, zi1, ue_pad, zu1)
    ua, ia, ug, ig = _tc_bcast(um_pad, vm_pad, su, si, zu1, zu2, zi1, zi2)
    return (ua[:NU], ia[:NI], ug[:NU], ig[:NI])


# speedup vs baseline: 8.4335x; 1.0191x over previous
import functools

import jax
import jax.numpy as jnp
from jax import lax
from jax.experimental import pallas as pl
from jax.experimental.pallas import tpu as pltpu
from jax.experimental.pallas import tpu_sc as plsc

NU = 25000
NI = 25000
D = 64
RK = 32
NNZ = 800000

NTILE = 16
NCORE = 2
NPAD = 25088
ROWS_PER_TILE = NPAD // NTILE
ZCHUNK = 16
CHUNK = 128
EPT = 50176
NNZ_PAD = EPT * NTILE
NCHUNK = EPT // CHUNK
BLKC = 8
NBLK = NCHUNK // BLKC
NBUF = 3
NIDX = 4
NVAL = 2


def _spmm_body(ego_u, ego_i, vals, rows, cols, zu_out, zi_out,
               gidxc, sidxc, valbc, gbufs, zbuf, accum,
               gsems, ssems, xsems, isems, vsems, zsem):
    cid = lax.axis_index("c")
    sid = lax.axis_index("s")

    zero16 = jnp.zeros((16,), jnp.float32)

    def zrow(e, carry):
        for j in range(4):
            zbuf[e, pl.ds(j * 16, 16)] = zero16
        return carry

    lax.fori_loop(0, ZCHUNK, zrow, 0)
    zds = []
    for k in range(ROWS_PER_TILE // ZCHUNK):
        zds.append(pltpu.async_copy(
            zbuf, accum.at[pl.ds(sid * ROWS_PER_TILE + k * ZCHUNK, ZCHUNK)],
            zsem))
    for d in zds:
        d.wait()
    plsc.subcore_barrier()

    def run_side(src_tbl, g_hbm, s_hbm, out_hbm):
        def block_body(b, carry):
            ebase = (sid * NCHUNK + b * BLKC) * CHUNK

            def load_idx(j):
                gx = pltpu.async_copy(
                    g_hbm.at[pl.ds(ebase + j * CHUNK, CHUNK)],
                    gidxc[j % NBUF], xsems[j % NBUF])
                si = pltpu.async_copy(
                    s_hbm.at[pl.ds(ebase + j * CHUNK, CHUNK)],
                    sidxc[j % NIDX], isems[j % NIDX])
                return gx, si

            gxds = [None] * BLKC
            ids = [None] * BLKC
            vds = [None] * BLKC
            gds = [None] * BLKC
            sds = [None] * BLKC
            gxds[0], ids[0] = load_idx(0)
            for j in range(BLKC + 2):
                if j >= 3 and j - 3 < BLKC:
                    sds[j - 3].wait()
                if j >= 2:
                    i = j - 2
                    buf = gbufs[i % NBUF]
                    vds[i].wait()
                    ids[i].wait()
                    gds[i].wait()
                    vb = valbc[i % NVAL]

                    def group_body(g, c2):
                        for t in range(0, 16, 8):
                            e0 = g * 16 + t
                            vv = [vb[e0 + k] for k in range(8)]
                            for q in range(4):
                                sl = pl.ds(q * 16, 16)
                                for k in range(8):
                                    buf[e0 + k, sl] = buf[e0 + k, sl] * vv[k]
                        return c2

                    lax.fori_loop(0, CHUNK // 16, group_body, 0)
                    sds[i] = pltpu.async_copy(buf, accum.at[sidxc[i % NIDX]],
                                              ssems[i % NBUF], add=True)
                if j < BLKC:
                    gxds[j].wait()
                    gds[j] = pltpu.async_copy(
                        src_tbl.at[gidxc[j % NBUF]], gbufs[j % NBUF],
                        gsems[j % NBUF])
                    vds[j] = pltpu.async_copy(
                        vals.at[pl.ds(ebase + j * CHUNK, CHUNK)],
                        valbc[j % NVAL], vsems[j % NVAL])
                if j + 1 < BLKC:
                    gxds[j + 1], ids[j + 1] = load_idx(j + 1)
            sds[BLKC - 1].wait()
            return carry

        lax.fori_loop(0, NBLK, block_body, 0)
        plsc.subcore_barrier()
        cpd = pltpu.async_copy(
            accum.at[pl.ds(sid * ROWS_PER_TILE, ROWS_PER_TILE)],
            out_hbm.at[pl.ds(sid * ROWS_PER_TILE, ROWS_PER_TILE)], zsem)
        cpd.wait()

    @pl.when(cid == 0)
    def _():
        run_side(ego_i, cols, rows, zu_out)

    @pl.when(cid == 1)
    def _():
        run_side(ego_u, rows, cols, zi_out)


@jax.jit
def _spmm(ego_u_pad, ego_i_pad, vals_pad, rows_pad, cols_pad):
    mesh = plsc.VectorSubcoreMesh(core_axis_name="c", subcore_axis_name="s",
                                  num_cores=NCORE, num_subcores=NTILE)
    f = pl.kernel(
        _spmm_body,
        out_type=[jax.ShapeDtypeStruct((NPAD, D), jnp.float32),
                  jax.ShapeDtypeStruct((NPAD, D), jnp.float32)],
        mesh=mesh,
        scratch_types=[
            [pltpu.VMEM((CHUNK,), jnp.int32)] * NBUF,
            [pltpu.VMEM((CHUNK,), jnp.int32)] * NIDX,
            [pltpu.VMEM((CHUNK, 16), jnp.float32)] * NVAL,
            [pltpu.VMEM((CHUNK, D), jnp.float32)] * NBUF,
            pltpu.VMEM((ZCHUNK, D), jnp.float32),
            pltpu.VMEM_SHARED((NPAD, D), jnp.float32),
            [pltpu.SemaphoreType.DMA] * NBUF,
            [pltpu.SemaphoreType.DMA] * NBUF,
            [pltpu.SemaphoreType.DMA] * NBUF,
            [pltpu.SemaphoreType.DMA] * NIDX,
            [pltpu.SemaphoreType.DMA] * NVAL,
            pltpu.SemaphoreType.DMA,
        ],
        compiler_params=pltpu.CompilerParams(use_tc_tiling_on_sc=False),
        name="lightgcl_spmm",
    )
    return f(ego_u_pad, ego_i_pad, vals_pad, rows_pad, cols_pad)


KBLK = 1792


def _reduce_body(vt_ref, ut_ref, ie_ref, zi1_ref, ue_ref, zu1_ref, su_ref, si_ref):
    @pl.when(pl.program_id(0) == 0)
    def _():
        su_ref[...] = jnp.zeros_like(su_ref)
        si_ref[...] = jnp.zeros_like(si_ref)

    su_ref[...] += jnp.dot(vt_ref[...], (ie_ref[...] + zi1_ref[...]) * 0.5,
                           preferred_element_type=jnp.float32)
    si_ref[...] += jnp.dot(ut_ref[...], (ue_ref[...] + zu1_ref[...]) * 0.5,
                           preferred_element_type=jnp.float32)


@jax.jit
def _tc_reduce(vt_pad, ut_pad, ie_pad, zi1, ue_pad, zu1):
    grid = NPAD // KBLK
    return pl.pallas_call(
        _reduce_body,
        grid=(grid,),
        in_specs=[
            pl.BlockSpec((RK, KBLK), lambda k: (0, k)),
            pl.BlockSpec((RK, KBLK), lambda k: (0, k)),
            pl.BlockSpec((KBLK, D), lambda k: (k, 0)),
            pl.BlockSpec((KBLK, D), lambda k: (k, 0)),
            pl.BlockSpec((KBLK, D), lambda k: (k, 0)),
            pl.BlockSpec((KBLK, D), lambda k: (k, 0)),
        ],
        out_specs=[
            pl.BlockSpec((RK, D), lambda k: (0, 0)),
            pl.BlockSpec((RK, D), lambda k: (0, 0)),
        ],
        out_shape=[jax.ShapeDtypeStruct((RK, D), jnp.float32),
                   jax.ShapeDtypeStruct((RK, D), jnp.float32)],
    )(vt_pad, ut_pad, ie_pad, zi1, ue_pad, zu1)


def _bcast_body(um_ref, vm_ref, su_ref, si_ref, zu1_ref, zu2_ref, zi1_ref, zi2_ref,
                ua_ref, ia_ref, ug_ref, ig_ref):
    ua_ref[...] = (zu1_ref[...] + zu2_ref[...]) * 0.5
    ia_ref[...] = (zi1_ref[...] + zi2_ref[...]) * 0.5
    ug_ref[...] = jnp.dot(um_ref[...], su_ref[...],
                          preferred_element_type=jnp.float32)
    ig_ref[...] = jnp.dot(vm_ref[...], si_ref[...],
                          preferred_element_type=jnp.float32)


@jax.jit
def _tc_bcast(um_pad, vm_pad, su, si, zu1, zu2, zi1, zi2):
    grid = NPAD // KBLK
    node_spec = pl.BlockSpec((KBLK, D), lambda k: (k, 0))
    full_spec = pl.BlockSpec((RK, D), lambda k: (0, 0))
    return pl.pallas_call(
        _bcast_body,
        grid=(grid,),
        in_specs=[
            pl.BlockSpec((KBLK, RK), lambda k: (k, 0)),
            pl.BlockSpec((KBLK, RK), lambda k: (k, 0)),
            full_spec, full_spec,
            node_spec, node_spec, node_spec, node_spec,
        ],
        out_specs=[node_spec, node_spec, node_spec, node_spec],
        out_shape=[jax.ShapeDtypeStruct((NPAD, D), jnp.float32)] * 4,
    )(um_pad, vm_pad, su, si, zu1, zu2, zi1, zi2)


def kernel(user_emb, item_emb, adj_vals, u_mul_s, v_mul_s, ut, vt, adj_rows, adj_cols):
    ue_pad = jnp.pad(user_emb, ((0, NPAD - NU), (0, 0)))
    ie_pad = jnp.pad(item_emb, ((0, NPAD - NI), (0, 0)))
    vals_pad = jnp.pad(adj_vals, (0, NNZ_PAD - NNZ))
    rows_pad = jnp.pad(adj_rows, (0, NNZ_PAD - NNZ))
    cols_pad = jnp.pad(adj_cols, (0, NNZ_PAD - NNZ))
    vt_pad = jnp.pad(vt, ((0, 0), (0, NPAD - NI)))
    ut_pad = jnp.pad(ut, ((0, 0), (0, NPAD - NU)))
    um_pad = jnp.pad(u_mul_s, ((0, NPAD - NU), (0, 0)))
    vm_pad = jnp.pad(v_mul_s, ((0, NPAD - NI), (0, 0)))

    vals_rep = jnp.broadcast_to(vals_pad[:, None], (NNZ_PAD, 16))

    zu1, zi1 = _spmm(ue_pad, ie_pad, vals_rep, rows_pad, cols_pad)
    zu2, zi2 = _spmm(zu1, zi1, vals_rep, rows_pad, cols_pad)
    su, si = _tc_reduce(vt_pad, ut_pad, ie_pad, zi1, ue_pad, zu1)
    ua, ia, ug, ig = _tc_bcast(um_pad, vm_pad, su, si, zu1, zu2, zi1, zi2)
    return (ua[:NU], ia[:NI], ug[:NU], ig[:NI])
